# baseline (speedup 1.0000x reference)
import jax
import jax.numpy as jnp
from jax.experimental import pallas as pl
from jax.experimental.pallas import tpu as pltpu

_MIN_NORM = 1e-15
_BALL_EPS = 4e-3
_HALF_LN2 = 0.34657359027997264

_INV_PI = 0.3183098861837907
_PI1 = 3.140625
_PI2 = 9.675025939941406e-4
_PI3 = 1.509957990978376e-7


def _cs(x):
    return jnp.sum(x, axis=0, keepdims=True)


def _fast_cos(x):
    qi = jnp.round(x * _INV_PI).astype(jnp.int32)
    qf = qi.astype(jnp.float32)
    r = x - qf * _PI1
    r = r - qf * _PI2
    r = r - qf * _PI3
    z = r * r
    p = 2.08767569878681e-9
    p = p * z - 2.7557319223985893e-7
    p = p * z + 2.48015873015873e-5
    p = p * z - 1.3888888888888887e-3
    p = p * z + 4.1666666666666664e-2
    val = (p * z - 0.5) * z + 1.0
    sign = (qi & 1) << 31
    bits = pltpu.bitcast(val, jnp.int32) ^ sign
    return pltpu.bitcast(bits, jnp.float32)


def _norm_inv(x):
    ss = jnp.maximum(_cs(x * x), _MIN_NORM * _MIN_NORM)
    rn = jax.lax.rsqrt(ss)
    return ss, ss * rn, rn


def _aon(n, inv_n):
    nc = jnp.minimum(n, 1.0 - 1e-7)
    t = (1.0 + nc) / (1.0 - nc)
    return (_HALF_LN2 * inv_n) * jnp.log2(t)


def _gru_body(mi_ref, mem_ref, ts_ref, mts_ref, h_ref, twb_ref, tbb_ref,
              wih_ref, whh_ref, biasT_ref, bbr_ref, bbh_ref, bbz_ref,
              nw_ref, nbb_ref, out_ref):
    f32 = jnp.float32
    hp = jax.lax.Precision.HIGHEST
    mi = mi_ref[...].T
    hx = mem_ref[...].T
    H = hx.shape[0]

    dt = ts_ref[...] - mts_ref[...]
    ang = dt * twb_ref[...] + tbb_ref[...]
    u = _fast_cos(ang)
    _, un, inv_un = _norm_inv(u)
    tn = jnp.tanh(un)
    maxnorm = 1.0 - _BALL_EPS
    pscale = jnp.where(tn > maxnorm, maxnorm / tn, 1.0)
    tf = (tn * inv_un * pscale) * u
    tfn = jnp.minimum(tn, maxnorm)

    mi2 = _cs(mi * mi)
    xss = jnp.maximum(mi2 + tfn * tfn, _MIN_NORM * _MIN_NORM)
    inv_xn = jax.lax.rsqrt(xss)
    xn = xss * inv_xn
    hss, hn, inv_hn = _norm_inv(hx)
    aox = _aon(xn, inv_xn)
    aoh = _aon(hn, inv_hn)

    xT = jnp.concatenate([mi, tf], axis=0)
    ux_all = jnp.dot(wih_ref[...], xT, preferred_element_type=f32)
    wh_all = jnp.dot(whh_ref[...], hx, preferred_element_type=f32)

    def mmv_post(m, aon_src):
        _, mxn, inv_mxn = _norm_inv(m)
        t = jnp.tanh(aon_src * mxn)
        return (t * inv_mxn) * m, t

    ux_r, t_uxr = mmv_post(ux_all[0:H], aox)
    ux_h, t_uxh = mmv_post(ux_all[H:2 * H], aox)
    ux_z, t_uxz = mmv_post(ux_all[2 * H:3 * H], aox)
    wh_r, t_whr = mmv_post(wh_all[0:H], aoh)
    wh_z, t_whz = mmv_post(wh_all[2 * H:3 * H], aoh)

    def madd(x, y, x2=None, y2=None):
        if x2 is None:
            x2 = _cs(x * x)
        if y2 is None:
            y2 = _cs(y * y)
        xy = _cs(x * y)
        num = (1.0 + 2.0 * xy + y2) * x + (1.0 - x2) * y
        inv_den = 1.0 / jnp.maximum(1.0 + 2.0 * xy + x2 * y2, _MIN_NORM)
        return num * inv_den

    biasT = biasT_ref[...]
    b2_all = _cs(biasT * biasT)
    bb_r, b_r2 = bbr_ref[...], b2_all[:, 0:1]
    bb_h, b_h2 = bbh_ref[...], b2_all[:, 1:2]
    bb_z, b_z2 = bbz_ref[...], b2_all[:, 2:3]

    gz = madd(madd(wh_z, ux_z, x2=t_whz * t_whz, y2=t_uxz * t_uxz), bb_z, y2=b_z2)
    gr = madd(madd(wh_r, ux_r, x2=t_whr * t_whr, y2=t_uxr * t_uxr), bb_r, y2=b_r2)

    def logmap_sig(y):
        _, n, inv_n = _norm_inv(y)
        return jax.nn.sigmoid(_aon(n, inv_n) * y)

    z = logmap_sig(gz)
    r = logmap_sig(gr)

    def mpm(w, x, aon_x):
        wx = w * x
        _, wxn, inv_wxn = _norm_inv(wx)
        t = jnp.tanh(aon_x * wxn)
        return (t * inv_wxn) * wx, t

    rh, t_rh = mpm(r, hx, aoh)
    rhn = jnp.maximum(t_rh, _MIN_NORM)
    aorh = _aon(rhn, 1.0 / rhn)
    wh_h = jnp.dot(whh_ref[H:2 * H, :], rh, preferred_element_type=f32)
    wh_h, t_whh = mmv_post(wh_h, aorh)
    h_tilde = madd(madd(wh_h, ux_h, x2=t_whh * t_whh, y2=t_uxh * t_uxh), bb_h, y2=b_h2)

    delta = madd(-hx, h_tilde, x2=hss)
    _, dn, inv_dn = _norm_inv(delta)
    aod = _aon(dn, inv_dn)
    zd, t_zd = mpm(z, delta, aod)
    upd = madd(hx, zd, x2=hss, y2=t_zd * t_zd)

    hm = jnp.dot(nw_ref[...], h_ref[...].T, preferred_element_type=f32) + nbb_ref[...]
    out_ref[...] = madd(upd, hm).T


def kernel(mem_input, mem, ts, mem_ts, h, time_w, time_b,
           weight_ih, weight_hh, bias, node_W, node_b):
    N, D_in = mem_input.shape
    H = mem.shape[1]
    D_node = h.shape[1]
    D_t = time_w.shape[0]

    B = 2048
    grid = (N // B,)

    ts_r = ts[None, :]
    mts_r = mem_ts[None, :]
    twb = jnp.broadcast_to(time_w[:, None], (D_t, B))
    tbb = jnp.broadcast_to(time_b[:, None], (D_t, B))
    biasT = bias.T
    bbr = jnp.broadcast_to(bias[0][:, None], (H, B))
    bbh = jnp.broadcast_to(bias[1][:, None], (H, B))
    bbz = jnp.broadcast_to(bias[2][:, None], (H, B))
    nbb = jnp.broadcast_to(node_b[:, None], (H, B))

    fixed = lambda i: (0, 0)
    cols = lambda i: (0, i)
    rows = lambda i: (i, 0)

    return pl.pallas_call(
        _gru_body,
        grid=grid,
        in_specs=[
            pl.BlockSpec((B, D_in), rows),
            pl.BlockSpec((B, H), rows),
            pl.BlockSpec((1, B), cols),
            pl.BlockSpec((1, B), cols),
            pl.BlockSpec((B, D_node), rows),
            pl.BlockSpec((D_t, B), fixed),
            pl.BlockSpec((D_t, B), fixed),
            pl.BlockSpec((3 * H, D_in + D_t), fixed),
            pl.BlockSpec((3 * H, H), fixed),
            pl.BlockSpec((H, 3), fixed),
            pl.BlockSpec((H, B), fixed),
            pl.BlockSpec((H, B), fixed),
            pl.BlockSpec((H, B), fixed),
            pl.BlockSpec((H, D_node), fixed),
            pl.BlockSpec((H, B), fixed),
        ],
        out_specs=pl.BlockSpec((B, H), rows),
        out_shape=jax.ShapeDtypeStruct((N, H), jnp.float32),
        compiler_params=pltpu.CompilerParams(
            dimension_semantics=("parallel",),
        ),
    )(mem_input, mem, ts_r, mts_r, h, twb, tbb, weight_ih, weight_hh, biasT,
      bbr, bbh, bbz, node_W, nbb)
